# baseline (speedup 1.0000x reference)
import functools

import jax
import jax.numpy as jnp
from jax import lax
from jax.experimental import pallas as pl
from jax.experimental.pallas import tpu as pltpu

_K = 20
_KS = 16
_ROWS = 256


def _round_bf16(v):
    u = lax.bitcast_convert_type(v, jnp.int32)
    lsb = jnp.bitwise_and(lax.shift_right_logical(u, 16), 1)
    r = jnp.bitwise_and(u + 0x7FFF + lsb, jnp.int32(-65536))
    return lax.bitcast_convert_type(r, jnp.float32)


def _pai_kernel(xt_row_ref, xt_all_ref, xlimb_ref, xxc_ref, xxl_ref,
                kern_ref, pad_ref, idx_ref, adj_ref, g_scr, work_scr):
    b = pl.program_id(0)
    xa = xt_all_ref[0]
    xlimb = xlimb_ref[0]
    xrq = _round_bf16(xt_row_ref[0])
    xaq = _round_bf16(xa)
    n = xa.shape[0]
    r = xrq.shape[0]
    f = xa.shape[1]

    inner = -2.0 * lax.dot_general(
        xrq, xaq, (((1,), (1,)), ((), ())),
        preferred_element_type=jnp.float32,
        precision=lax.Precision.HIGHEST,
    )
    pd = (-xxl_ref[0] - inner) - xxc_ref[0]

    col = lax.broadcasted_iota(jnp.int32, pd.shape, 1)
    kcol = lax.broadcasted_iota(jnp.int32, (r, _K), 1)
    neg = jnp.float32(-jnp.inf)

    work_scr[...] = pd

    def topk_body(carry):
        j, idxm = carry
        work = work_scr[...]
        m = jnp.max(work, axis=1, keepdims=True)
        eq = work == m
        idxj = jnp.min(jnp.where(eq, col, n), axis=1, keepdims=True)
        onehot = col == idxj
        work_scr[...] = jnp.where(onehot, neg, work)
        idxm = jnp.where(kcol == j, idxj, idxm)
        oh_bf = jnp.where(onehot, 1.0, 0.0).astype(jnp.bfloat16)
        gj = lax.dot_general(
            oh_bf, xlimb, (((1,), (0,)), ((), ())),
            preferred_element_type=jnp.float32,
        )
        sj = (gj[:, :f] + gj[:, f:2 * f]) + gj[:, 2 * f:]
        g_scr[pl.ds(j, 1)] = sj[None]
        return j + 1, idxm

    idxm0 = jnp.zeros((r, _K), jnp.int32)
    _, idxm = lax.while_loop(lambda c: c[0] < _K, topk_body, (0, idxm0))
    idx_ref[0] = idxm + b * n

    kern_q = _round_bf16(kern_ref[...])
    s0 = g_scr[0]
    w_list = []
    for j in range(_K):
        dsq = _round_bf16(g_scr[j] - s0)
        aj = lax.dot_general(
            dsq, kern_q, (((1,), (0,)), ((), ())),
            preferred_element_type=jnp.float32,
            precision=lax.Precision.HIGHEST,
        ) + pad_ref[j]
        w_list.append(jnp.where(aj > 0, aj, 0.0))

    tot = functools.reduce(jnp.add, w_list) + 1e-06
    w_list = [w / tot for w in w_list]
    w_list = [w * w for w in w_list]
    tot2 = functools.reduce(jnp.add, w_list) + 1e-06
    for j in range(_K):
        wj = w_list[j] / tot2
        adj_ref[0, :, j, :] = jnp.where(wj > 0.1, wj, 0.0)


def kernel(x, kernals, one_padding):
    bsize, feats, n = x.shape
    xt = jnp.transpose(x, (0, 2, 1))
    xx = jnp.sum(x ** 2, axis=1)
    h = _round_bf16(xt)
    rm = xt - h
    m = _round_bf16(rm)
    l = rm - m
    xlimb = jnp.concatenate(
        [h.astype(jnp.bfloat16), m.astype(jnp.bfloat16),
         l.astype(jnp.bfloat16)], axis=2)
    idx, adj = pl.pallas_call(
        _pai_kernel,
        grid=(bsize, n // _ROWS),
        in_specs=[
            pl.BlockSpec((1, _ROWS, feats), lambda b, r: (b, r, 0)),
            pl.BlockSpec((1, n, feats), lambda b, r: (b, 0, 0)),
            pl.BlockSpec((1, n, 3 * feats), lambda b, r: (b, 0, 0)),
            pl.BlockSpec((1, _ROWS, 1), lambda b, r: (b, r, 0)),
            pl.BlockSpec((1, 1, n), lambda b, r: (b, 0, 0)),
            pl.BlockSpec((feats, _KS), lambda b, r: (0, 0)),
            pl.BlockSpec((_K, _KS), lambda b, r: (0, 0)),
        ],
        out_specs=(
            pl.BlockSpec((1, _ROWS, _K), lambda b, r: (b, r, 0)),
            pl.BlockSpec((1, _ROWS, _K, _KS), lambda b, r: (b, r, 0, 0)),
        ),
        out_shape=(
            jax.ShapeDtypeStruct((bsize, n, _K), jnp.int32),
            jax.ShapeDtypeStruct((bsize, n, _K, _KS), jnp.float32),
        ),
        scratch_shapes=[
            pltpu.VMEM((_K, _ROWS, feats), jnp.float32),
            pltpu.VMEM((_ROWS, n), jnp.float32),
        ],
    )(xt, xt, xlimb, xx[:, :, None], xx[:, None, :], kernals, one_padding)
    return idx.reshape(-1), adj.reshape(bsize * n, _K, _KS)


# speedup vs baseline: 1.0515x; 1.0515x over previous
import functools

import jax
import jax.numpy as jnp
from jax import lax
from jax.experimental import pallas as pl
from jax.experimental.pallas import tpu as pltpu

_K = 20
_KS = 16
_ROWS = 256


def _round_bf16(v):
    u = lax.bitcast_convert_type(v, jnp.int32)
    lsb = jnp.bitwise_and(lax.shift_right_logical(u, 16), 1)
    r = jnp.bitwise_and(u + 0x7FFF + lsb, jnp.int32(-65536))
    return lax.bitcast_convert_type(r, jnp.float32)


def _pai_kernel(xt_row_ref, xt_all_ref, xlimb_ref, xxc_ref, xxl_ref,
                kern_ref, pad_ref, idx_ref, adj_ref, g_scr, work_scr):
    b = pl.program_id(0)
    xa = xt_all_ref[0]
    xlimb = xlimb_ref[0]
    xrq = _round_bf16(xt_row_ref[0])
    xaq = _round_bf16(xa)
    n = xa.shape[0]
    r = xrq.shape[0]
    f = xa.shape[1]

    inner = -2.0 * lax.dot_general(
        xrq, xaq, (((1,), (1,)), ((), ())),
        preferred_element_type=jnp.float32,
        precision=lax.Precision.HIGHEST,
    )
    pd = (-xxl_ref[0] - inner) - xxc_ref[0]

    col = lax.broadcasted_iota(jnp.int32, pd.shape, 1)
    kcol = lax.broadcasted_iota(jnp.int32, (r, _K), 1)
    neg = jnp.float32(-jnp.inf)

    work_scr[...] = pd

    def topk_body(carry):
        j, idxm = carry
        work = work_scr[...]
        m = jnp.max(work, axis=1, keepdims=True)
        eq = work == m
        idxj = jnp.min(jnp.where(eq, col, n), axis=1, keepdims=True)
        onehot = col == idxj
        work_scr[...] = jnp.where(onehot, neg, work)
        idxm = jnp.where(kcol == j, idxj, idxm)
        oh_bf = jnp.where(onehot, 1.0, 0.0).astype(jnp.bfloat16)
        gj = lax.dot_general(
            oh_bf, xlimb, (((1,), (0,)), ((), ())),
            preferred_element_type=jnp.float32,
        )
        sj = (gj[:, :f] + gj[:, f:2 * f]) + gj[:, 2 * f:]
        g_scr[pl.ds(j, 1)] = sj[None]
        return j + 1, idxm

    idxm0 = jnp.zeros((r, _K), jnp.int32)
    _, idxm = lax.while_loop(lambda c: c[0] < _K, topk_body, (0, idxm0))
    idx_ref[0] = idxm + b * n

    kern_bf = _round_bf16(kern_ref[...]).astype(jnp.bfloat16)
    s0 = g_scr[0]
    w_list = []
    for j in range(_K):
        dsq = _round_bf16(g_scr[j] - s0).astype(jnp.bfloat16)
        aj = lax.dot_general(
            dsq, kern_bf, (((1,), (0,)), ((), ())),
            preferred_element_type=jnp.float32,
        ) + pad_ref[j]
        w_list.append(jnp.where(aj > 0, aj, 0.0))

    tot = functools.reduce(jnp.add, w_list) + 1e-06
    w_list = [w / tot for w in w_list]
    w_list = [w * w for w in w_list]
    tot2 = functools.reduce(jnp.add, w_list) + 1e-06
    for j in range(_K):
        wj = w_list[j] / tot2
        adj_ref[0, :, j, :] = jnp.where(wj > 0.1, wj, 0.0)


def kernel(x, kernals, one_padding):
    bsize, feats, n = x.shape
    xt = jnp.transpose(x, (0, 2, 1))
    xx = jnp.sum(x ** 2, axis=1)
    h = _round_bf16(xt)
    rm = xt - h
    m = _round_bf16(rm)
    l = rm - m
    xlimb = jnp.concatenate(
        [h.astype(jnp.bfloat16), m.astype(jnp.bfloat16),
         l.astype(jnp.bfloat16)], axis=2)
    idx, adj = pl.pallas_call(
        _pai_kernel,
        grid=(bsize, n // _ROWS),
        in_specs=[
            pl.BlockSpec((1, _ROWS, feats), lambda b, r: (b, r, 0)),
            pl.BlockSpec((1, n, feats), lambda b, r: (b, 0, 0)),
            pl.BlockSpec((1, n, 3 * feats), lambda b, r: (b, 0, 0)),
            pl.BlockSpec((1, _ROWS, 1), lambda b, r: (b, r, 0)),
            pl.BlockSpec((1, 1, n), lambda b, r: (b, 0, 0)),
            pl.BlockSpec((feats, _KS), lambda b, r: (0, 0)),
            pl.BlockSpec((_K, _KS), lambda b, r: (0, 0)),
        ],
        out_specs=(
            pl.BlockSpec((1, _ROWS, _K), lambda b, r: (b, r, 0)),
            pl.BlockSpec((1, _ROWS, _K, _KS), lambda b, r: (b, r, 0, 0)),
        ),
        out_shape=(
            jax.ShapeDtypeStruct((bsize, n, _K), jnp.int32),
            jax.ShapeDtypeStruct((bsize, n, _K, _KS), jnp.float32),
        ),
        scratch_shapes=[
            pltpu.VMEM((_K, _ROWS, feats), jnp.float32),
            pltpu.VMEM((_ROWS, n), jnp.float32),
        ],
    )(xt, xt, xlimb, xx[:, :, None], xx[:, None, :], kernals, one_padding)
    return idx.reshape(-1), adj.reshape(bsize * n, _K, _KS)


# speedup vs baseline: 1.1158x; 1.0611x over previous
import functools

import jax
import jax.numpy as jnp
from jax import lax
from jax.experimental import pallas as pl
from jax.experimental.pallas import tpu as pltpu

_K = 20
_KS = 16
_ROWS = 256


def _round_bf16(v):
    u = lax.bitcast_convert_type(v, jnp.int32)
    lsb = jnp.bitwise_and(lax.shift_right_logical(u, 16), 1)
    r = jnp.bitwise_and(u + 0x7FFF + lsb, jnp.int32(-65536))
    return lax.bitcast_convert_type(r, jnp.float32)


def _pai_kernel(xt_row_ref, xt_all_ref, xlimb_ref, xxc_ref, xxl_ref,
                kern_ref, pad_ref, idx_ref, adj_ref, g_scr, work_scr):
    b = pl.program_id(0)
    xa = xt_all_ref[0]
    xlimb = xlimb_ref[0]
    xrq = xt_row_ref[0].astype(jnp.bfloat16)
    xaq = xa.astype(jnp.bfloat16)
    n = xa.shape[0]
    r = xrq.shape[0]
    f = xa.shape[1]

    inner = -2.0 * lax.dot_general(
        xrq, xaq, (((1,), (1,)), ((), ())),
        preferred_element_type=jnp.float32,
    )
    pd = (-xxl_ref[0] - inner) - xxc_ref[0]

    col = lax.broadcasted_iota(jnp.int32, pd.shape, 1)
    kcol = lax.broadcasted_iota(jnp.int32, (r, _K), 1)
    neg = jnp.float32(-jnp.inf)

    work_scr[...] = pd

    def topk_body(carry):
        j, idxm = carry
        work = work_scr[...]
        m = jnp.max(work, axis=1, keepdims=True)
        eq = work == m
        idxj = jnp.min(jnp.where(eq, col, n), axis=1, keepdims=True)
        onehot = col == idxj
        work_scr[...] = jnp.where(onehot, neg, work)
        idxm = jnp.where(kcol == j, idxj, idxm)
        oh_bf = jnp.where(onehot, 1.0, 0.0).astype(jnp.bfloat16)
        gj = lax.dot_general(
            oh_bf, xlimb, (((1,), (0,)), ((), ())),
            preferred_element_type=jnp.float32,
        )
        sj = (gj[:, :f] + gj[:, f:2 * f]) + gj[:, 2 * f:]
        g_scr[pl.ds(j, 1)] = sj[None]
        return j + 1, idxm

    idxm0 = jnp.zeros((r, _K), jnp.int32)
    _, idxm = lax.while_loop(lambda c: c[0] < _K, topk_body, (0, idxm0))
    idx_ref[0] = idxm + b * n

    kern_bf = kern_ref[...].astype(jnp.bfloat16)
    s0 = g_scr[0]
    w_list = []
    for j in range(_K):
        dsq = (g_scr[j] - s0).astype(jnp.bfloat16)
        aj = lax.dot_general(
            dsq, kern_bf, (((1,), (0,)), ((), ())),
            preferred_element_type=jnp.float32,
        ) + pad_ref[j]
        w_list.append(jnp.where(aj > 0, aj, 0.0))

    tot = functools.reduce(jnp.add, w_list) + 1e-06
    w_list = [w / tot for w in w_list]
    w_list = [w * w for w in w_list]
    tot2 = functools.reduce(jnp.add, w_list) + 1e-06
    for j in range(_K):
        wj = w_list[j] / tot2
        adj_ref[0, :, j, :] = jnp.where(wj > 0.1, wj, 0.0)


def kernel(x, kernals, one_padding):
    bsize, feats, n = x.shape
    xt = jnp.transpose(x, (0, 2, 1))
    xx = jnp.sum(x ** 2, axis=1)
    h = _round_bf16(xt)
    rm = xt - h
    m = _round_bf16(rm)
    l = rm - m
    xlimb = jnp.concatenate(
        [h.astype(jnp.bfloat16), m.astype(jnp.bfloat16),
         l.astype(jnp.bfloat16)], axis=2)
    idx, adj = pl.pallas_call(
        _pai_kernel,
        grid=(bsize, n // _ROWS),
        in_specs=[
            pl.BlockSpec((1, _ROWS, feats), lambda b, r: (b, r, 0)),
            pl.BlockSpec((1, n, feats), lambda b, r: (b, 0, 0)),
            pl.BlockSpec((1, n, 3 * feats), lambda b, r: (b, 0, 0)),
            pl.BlockSpec((1, _ROWS, 1), lambda b, r: (b, r, 0)),
            pl.BlockSpec((1, 1, n), lambda b, r: (b, 0, 0)),
            pl.BlockSpec((feats, _KS), lambda b, r: (0, 0)),
            pl.BlockSpec((_K, _KS), lambda b, r: (0, 0)),
        ],
        out_specs=(
            pl.BlockSpec((1, _ROWS, _K), lambda b, r: (b, r, 0)),
            pl.BlockSpec((1, _ROWS, _K, _KS), lambda b, r: (b, r, 0, 0)),
        ),
        out_shape=(
            jax.ShapeDtypeStruct((bsize, n, _K), jnp.int32),
            jax.ShapeDtypeStruct((bsize, n, _K, _KS), jnp.float32),
        ),
        scratch_shapes=[
            pltpu.VMEM((_K, _ROWS, feats), jnp.float32),
            pltpu.VMEM((_ROWS, n), jnp.float32),
        ],
    )(xt, xt, xlimb, xx[:, :, None], xx[:, None, :], kernals, one_padding)
    return idx.reshape(-1), adj.reshape(bsize * n, _K, _KS)
